# baseline (speedup 1.0000x reference)
import functools

import jax
import jax.numpy as jnp
from jax import lax
from jax.experimental import pallas as pl
from jax.experimental.pallas import tpu as pltpu
from jax.experimental.pallas import tpu_sc as plsc

N = 10000
D = 128
E = 320000
NUM_SEEDS = 4500

NC = 2
NS = 16
L = 16

NROWS = 10240
ROWS_PER_TILE = NROWS // NS
ZROWS = 64
CHUNK = 128
KSUP = 16
CE = 160
NSC = CE // KSUP
EDGES_PER_TILE = CE * CHUNK
E_PAD = EDGES_PER_TILE * NS
NCHG = E_PAD // CHUNK
SEEDS_PER_TILE = 288
SEED_CHUNK = 96
CS = SEEDS_PER_TILE // SEED_CHUNK
SEEDS_PAD = SEEDS_PER_TILE * NS

BR = 1024
NBLK = NROWS // BR

_MESH = plsc.VectorSubcoreMesh(core_axis_name="c", subcore_axis_name="s")


def _sc_count(dst_idx):
    @functools.partial(
        pl.kernel,
        out_type=jax.ShapeDtypeStruct((NC, NROWS, D), jnp.float32),
        mesh=_MESH,
        scratch_types=[
            pltpu.VMEM((CHUNK,), jnp.int32),
            pltpu.VMEM((CHUNK, D), jnp.float32),
            pltpu.VMEM_SHARED((NROWS, D), jnp.float32),
        ],
    )
    def k(dst_hbm, cnt_hbm, idx_v, ones_v, cnt_sh):
        c = lax.axis_index("c")
        s = lax.axis_index("s")
        row0 = s * ROWS_PER_TILE
        ebase = c * E_PAD + s * EDGES_PER_TILE

        @pl.loop(0, CHUNK * (D // L))
        def _(t):
            r = t // (D // L)
            kk = t - r * (D // L)
            ones_v[r, pl.ds(kk * L, L)] = jnp.zeros((L,), jnp.float32)

        @pl.loop(0, ROWS_PER_TILE // CHUNK)
        def _(j):
            pltpu.sync_copy(ones_v, cnt_sh.at[pl.ds(row0 + j * CHUNK, CHUNK)])

        @pl.loop(0, CHUNK * (D // L))
        def _(t):
            r = t // (D // L)
            kk = t - r * (D // L)
            ones_v[r, pl.ds(kk * L, L)] = jnp.full((L,), 1.0, jnp.float32)

        plsc.subcore_barrier()

        @pl.loop(0, CE)
        def _(j):
            pltpu.sync_copy(dst_hbm.at[pl.ds(ebase + j * CHUNK, CHUNK)], idx_v)
            pltpu.sync_copy(ones_v, cnt_sh.at[idx_v], add=True)

        plsc.subcore_barrier()

        @pl.loop(0, ROWS_PER_TILE // CHUNK)
        def _(j):
            pltpu.sync_copy(cnt_sh.at[pl.ds(row0 + j * CHUNK, CHUNK)], ones_v)
            pltpu.sync_copy(ones_v, cnt_hbm.at[c, pl.ds(row0 + j * CHUNK, CHUNK)])

    return k(dst_idx)


def _sc_scatter(hn, idx2):
    @functools.partial(
        pl.kernel,
        out_type=jax.ShapeDtypeStruct((NC, NROWS, D), jnp.float32),
        mesh=_MESH,
        scratch_types=[
            pltpu.VMEM((2 * KSUP, CHUNK), jnp.int32),
            pltpu.VMEM((2 * KSUP, CHUNK), jnp.int32),
            pltpu.VMEM((CHUNK, D), jnp.float32),
            pltpu.VMEM((CHUNK, D), jnp.float32),
            pltpu.VMEM_SHARED((NROWS, D), jnp.float32),
            pltpu.SemaphoreType.DMA,
            pltpu.SemaphoreType.DMA,
            pltpu.SemaphoreType.DMA,
            pltpu.SemaphoreType.DMA,
            pltpu.SemaphoreType.DMA,
        ],
    )
    def k(hn_hbm, idx2_hbm, s_hbm, ib0, ib1, r0, r1,
          acc_sh, isem, gsem0, gsem1, ssem0, ssem1):
        c = lax.axis_index("c")
        s = lax.axis_index("s")
        row0 = s * ROWS_PER_TILE
        cbase = c * NCHG + s * CE
        rows = (r0, r1)
        gsems = (gsem0, gsem1)
        ssems = (ssem0, ssem1)
        ibufs = (ib0, ib1)

        @pl.loop(0, CHUNK * (D // L))
        def _(t):
            r = t // (D // L)
            kk = t - r * (D // L)
            r0[r, pl.ds(kk * L, L)] = jnp.zeros((L,), jnp.float32)

        @pl.loop(0, ROWS_PER_TILE // CHUNK)
        def _(j):
            pltpu.sync_copy(r0, acc_sh.at[pl.ds(row0 + j * CHUNK, CHUNK)])

        plsc.subcore_barrier()

        pltpu.async_copy(idx2_hbm.at[pl.ds(2 * cbase, 2 * KSUP)], ib0,
                         isem).wait()
        pltpu.async_copy(hn_hbm.at[c].at[ib0.at[0]], r0, gsem0)

        @pl.loop(0, NSC // 2)
        def _(q):
            for p in range(2):
                sc_first = (p == 0)
                ib_cur = ibufs[p]
                ib_nxt = ibufs[1 - p]
                for i in range(KSUP):
                    b = i % 2
                    nb = 1 - b
                    pltpu.make_async_copy(
                        hn_hbm.at[c].at[ib_cur.at[2 * i]], rows[b],
                        gsems[b]).wait()
                    pltpu.async_copy(rows[b],
                                     acc_sh.at[ib_cur.at[2 * i + 1]],
                                     ssems[b], add=True)

                    def _drain_prev():
                        pltpu.make_async_copy(
                            rows[nb], acc_sh.at[ib_cur.at[2 * i + 1]],
                            ssems[nb]).wait()
                    if sc_first and i == 0:
                        @pl.when(q > 0)
                        def _():
                            _drain_prev()
                    else:
                        _drain_prev()

                    if i == 0:
                        sc_next_base = cbase + (2 * q + p + 1) * KSUP

                        def _load_next():
                            pltpu.async_copy(
                                idx2_hbm.at[pl.ds(2 * sc_next_base, 2 * KSUP)],
                                ib_nxt, isem)
                        if p == 1:
                            @pl.when(q < NSC // 2 - 1)
                            def _():
                                _load_next()
                        else:
                            _load_next()

                    if i < KSUP - 1:
                        pltpu.async_copy(hn_hbm.at[c].at[ib_cur.at[2 * i + 2]],
                                         rows[nb], gsems[nb])
                    else:
                        def _next_sc_gather():
                            pltpu.make_async_copy(
                                idx2_hbm.at[pl.ds(2 * cbase, 2 * KSUP)],
                                ib_nxt, isem).wait()
                            pltpu.async_copy(hn_hbm.at[c].at[ib_nxt.at[0]],
                                             rows[nb], gsems[nb])
                        if p == 1:
                            @pl.when(q < NSC // 2 - 1)
                            def _():
                                _next_sc_gather()
                        else:
                            _next_sc_gather()

        pltpu.make_async_copy(rows[1], acc_sh.at[ib1.at[3]], ssems[1]).wait()

        plsc.subcore_barrier()

        @pl.loop(0, ROWS_PER_TILE // CHUNK)
        def _(j):
            pltpu.sync_copy(acc_sh.at[pl.ds(row0 + j * CHUNK, CHUNK)], r0)
            pltpu.sync_copy(r0, s_hbm.at[c, pl.ds(row0 + j * CHUNK, CHUNK)])

    return k(hn, idx2)


def _sc_seed_gather(h, seeds):
    @functools.partial(
        pl.kernel,
        out_type=jax.ShapeDtypeStruct((NC, SEEDS_PAD, D), jnp.float32),
        mesh=_MESH,
        scratch_types=[
            pltpu.VMEM((SEED_CHUNK,), jnp.int32),
            pltpu.VMEM((SEED_CHUNK, D), jnp.float32),
            pltpu.SemaphoreType.DMA,
        ],
    )
    def k(h_hbm, seeds_hbm, out_hbm, idx_v, rows_v, sem):
        c = lax.axis_index("c")
        s = lax.axis_index("s")

        @pl.loop(0, CS)
        def _(j):
            off = s * SEEDS_PER_TILE + j * SEED_CHUNK
            pltpu.sync_copy(seeds_hbm.at[pl.ds(c * SEEDS_PAD + off, SEED_CHUNK)], idx_v)
            pltpu.async_copy(h_hbm.at[c].at[idx_v], rows_v, sem).wait()
            pltpu.sync_copy(rows_v, out_hbm.at[c, pl.ds(off, SEED_CHUNK)])

    return k(h, seeds)


def _dinv_block(cnt_blk, blk_idx):
    dinv = lax.rsqrt(cnt_blk + 1.0)
    rows = blk_idx * BR + lax.broadcasted_iota(jnp.int32, (BR, 1), 0)
    return jnp.where(rows < N, dinv, 0.0)


def _tc_hn0(x, cnt):
    def body(x_ref, cnt_ref, hn_ref):
        dinv = _dinv_block(cnt_ref[0], pl.program_id(1))
        hn_ref[0] = x_ref[0] * dinv

    return pl.pallas_call(
        body,
        grid=(NC, NBLK),
        in_specs=[
            pl.BlockSpec((1, BR, D), lambda g, i: (g, i, 0)),
            pl.BlockSpec((1, BR, D), lambda g, i: (g, i, 0)),
        ],
        out_specs=pl.BlockSpec((1, BR, D), lambda g, i: (g, i, 0)),
        out_shape=jax.ShapeDtypeStruct((NC, NROWS, D), jnp.float32),
    )(x, cnt)


def _tc_layer(s_arr, hn, hp, cnt, w):
    def body(s_ref, hn_ref, hp_ref, cnt_ref, w_ref, ho_ref, hno_ref):
        dinv = _dinv_block(cnt_ref[0], pl.program_id(1))
        agg = (s_ref[0] + hn_ref[0]) * dinv
        h = jnp.dot(agg, w_ref[...], preferred_element_type=jnp.float32) + hp_ref[0]
        h = jnp.maximum(h, 0.0)
        ho_ref[0] = h
        hno_ref[0] = h * dinv

    return pl.pallas_call(
        body,
        grid=(NC, NBLK),
        in_specs=[
            pl.BlockSpec((1, BR, D), lambda g, i: (g, i, 0)),
            pl.BlockSpec((1, BR, D), lambda g, i: (g, i, 0)),
            pl.BlockSpec((1, BR, D), lambda g, i: (g, i, 0)),
            pl.BlockSpec((1, BR, D), lambda g, i: (g, i, 0)),
            pl.BlockSpec((D, D), lambda g, i: (0, 0)),
        ],
        out_specs=[
            pl.BlockSpec((1, BR, D), lambda g, i: (g, i, 0)),
            pl.BlockSpec((1, BR, D), lambda g, i: (g, i, 0)),
        ],
        out_shape=[
            jax.ShapeDtypeStruct((NC, NROWS, D), jnp.float32),
            jax.ShapeDtypeStruct((NC, NROWS, D), jnp.float32),
        ],
    )(s_arr, hn, hp, cnt, w)


def _pad_idx(a, n_pad, fill):
    return jnp.concatenate([a, jnp.full((n_pad - a.shape[0],), fill, jnp.int32)])


def kernel(sr_ent_seeds, tg_ent_seeds, edges_sr, edges_tg,
           entity_vec_sr, entity_vec_tg, W0, W1):
    src = jnp.concatenate([_pad_idx(edges_sr[0], E_PAD, N),
                           _pad_idx(edges_tg[0], E_PAD, N)])
    dst = jnp.concatenate([_pad_idx(edges_sr[1], E_PAD, N),
                           _pad_idx(edges_tg[1], E_PAD, N)])
    idx2 = jnp.stack([src.reshape(2 * NCHG, CHUNK),
                      dst.reshape(2 * NCHG, CHUNK)],
                     axis=1).reshape(4 * NCHG, CHUNK)
    seeds = jnp.concatenate([_pad_idx(sr_ent_seeds, SEEDS_PAD, 0),
                             _pad_idx(tg_ent_seeds, SEEDS_PAD, 0)])
    x0 = jnp.stack([
        jnp.pad(entity_vec_sr, ((0, NROWS - N), (0, 0))),
        jnp.pad(entity_vec_tg, ((0, NROWS - N), (0, 0))),
    ])

    cnt = _sc_count(dst)
    hn0 = _tc_hn0(x0, cnt)
    s0 = _sc_scatter(hn0, idx2)
    h1, hn1 = _tc_layer(s0, hn0, x0, cnt, W0)
    s1 = _sc_scatter(hn1, idx2)
    h2, _ = _tc_layer(s1, hn1, h1, cnt, W1)
    sg = _sc_seed_gather(h2, seeds)

    return (sg[0, :NUM_SEEDS], sg[1, :NUM_SEEDS], h2[0, :N], h2[1, :N])


# speedup vs baseline: 1.1168x; 1.1168x over previous
import functools

import jax
import jax.numpy as jnp
from jax import lax
from jax.experimental import pallas as pl
from jax.experimental.pallas import tpu as pltpu
from jax.experimental.pallas import tpu_sc as plsc

N = 10000
D = 128
E = 320000
NUM_SEEDS = 4500

NC = 2
NS = 16
L = 16

NROWS = 10240
ROWS_PER_TILE = NROWS // NS
ZROWS = 64
CHUNK = 128
EDGES_PER_TILE = 20096
CE = EDGES_PER_TILE // CHUNK
E_PAD = EDGES_PER_TILE * NS
SEEDS_PER_TILE = 288
SEED_CHUNK = 96
CS = SEEDS_PER_TILE // SEED_CHUNK
SEEDS_PAD = SEEDS_PER_TILE * NS

BR = 1024
NBLK = NROWS // BR

_MESH = plsc.VectorSubcoreMesh(core_axis_name="c", subcore_axis_name="s")


def _sc_count(dst_idx):
    @functools.partial(
        pl.kernel,
        out_type=jax.ShapeDtypeStruct((NC, NROWS, D), jnp.float32),
        mesh=_MESH,
        scratch_types=[
            pltpu.VMEM((CHUNK,), jnp.int32),
            pltpu.VMEM((CHUNK, D), jnp.float32),
            pltpu.VMEM_SHARED((NROWS, D), jnp.float32),
        ],
    )
    def k(dst_hbm, cnt_hbm, idx_v, ones_v, cnt_sh):
        c = lax.axis_index("c")
        s = lax.axis_index("s")
        row0 = s * ROWS_PER_TILE
        ebase = c * E_PAD + s * EDGES_PER_TILE

        @pl.loop(0, CHUNK * (D // L))
        def _(t):
            r = t // (D // L)
            kk = t - r * (D // L)
            ones_v[r, pl.ds(kk * L, L)] = jnp.zeros((L,), jnp.float32)

        @pl.loop(0, ROWS_PER_TILE // CHUNK)
        def _(j):
            pltpu.sync_copy(ones_v, cnt_sh.at[pl.ds(row0 + j * CHUNK, CHUNK)])

        @pl.loop(0, CHUNK * (D // L))
        def _(t):
            r = t // (D // L)
            kk = t - r * (D // L)
            ones_v[r, pl.ds(kk * L, L)] = jnp.full((L,), 1.0, jnp.float32)

        plsc.subcore_barrier()

        @pl.loop(0, CE)
        def _(j):
            pltpu.sync_copy(dst_hbm.at[pl.ds(ebase + j * CHUNK, CHUNK)], idx_v)
            pltpu.sync_copy(ones_v, cnt_sh.at[idx_v], add=True)

        plsc.subcore_barrier()

        @pl.loop(0, ROWS_PER_TILE // CHUNK)
        def _(j):
            pltpu.sync_copy(cnt_sh.at[pl.ds(row0 + j * CHUNK, CHUNK)], ones_v)
            pltpu.sync_copy(ones_v, cnt_hbm.at[c, pl.ds(row0 + j * CHUNK, CHUNK)])

    return k(dst_idx)


def _sc_scatter(hn, src_idx, dst_idx):
    @functools.partial(
        pl.kernel,
        out_type=jax.ShapeDtypeStruct((NC, NROWS, D), jnp.float32),
        mesh=_MESH,
        scratch_types=[
            pltpu.VMEM((CHUNK,), jnp.int32),
            pltpu.VMEM((CHUNK,), jnp.int32),
            pltpu.VMEM((CHUNK, D), jnp.float32),
            pltpu.VMEM_SHARED((NROWS, D), jnp.float32),
            pltpu.SemaphoreType.DMA,
        ],
    )
    def k(hn_hbm, src_hbm, dst_hbm, s_hbm, si_v, di_v, rows_v, acc_sh, sem):
        c = lax.axis_index("c")
        s = lax.axis_index("s")
        row0 = s * ROWS_PER_TILE
        ebase = c * E_PAD + s * EDGES_PER_TILE

        @pl.loop(0, CHUNK * (D // L))
        def _(t):
            r = t // (D // L)
            kk = t - r * (D // L)
            rows_v[r, pl.ds(kk * L, L)] = jnp.zeros((L,), jnp.float32)

        @pl.loop(0, ROWS_PER_TILE // CHUNK)
        def _(j):
            pltpu.sync_copy(rows_v,
                            acc_sh.at[pl.ds(row0 + j * CHUNK, CHUNK)])

        plsc.subcore_barrier()

        @pl.loop(0, CE)
        def _(j):
            pltpu.sync_copy(src_hbm.at[pl.ds(ebase + j * CHUNK, CHUNK)], si_v)
            pltpu.sync_copy(dst_hbm.at[pl.ds(ebase + j * CHUNK, CHUNK)], di_v)
            pltpu.async_copy(hn_hbm.at[c].at[si_v], rows_v, sem).wait()
            pltpu.sync_copy(rows_v, acc_sh.at[di_v], add=True)

        plsc.subcore_barrier()

        @pl.loop(0, ROWS_PER_TILE // CHUNK)
        def _(j):
            pltpu.sync_copy(acc_sh.at[pl.ds(row0 + j * CHUNK, CHUNK)], rows_v)
            pltpu.sync_copy(rows_v, s_hbm.at[c, pl.ds(row0 + j * CHUNK, CHUNK)])

    return k(hn, src_idx, dst_idx)


def _sc_seed_gather(h, seeds):
    @functools.partial(
        pl.kernel,
        out_type=jax.ShapeDtypeStruct((NC, SEEDS_PAD, D), jnp.float32),
        mesh=_MESH,
        scratch_types=[
            pltpu.VMEM((SEED_CHUNK,), jnp.int32),
            pltpu.VMEM((SEED_CHUNK, D), jnp.float32),
            pltpu.SemaphoreType.DMA,
        ],
    )
    def k(h_hbm, seeds_hbm, out_hbm, idx_v, rows_v, sem):
        c = lax.axis_index("c")
        s = lax.axis_index("s")

        @pl.loop(0, CS)
        def _(j):
            off = s * SEEDS_PER_TILE + j * SEED_CHUNK
            pltpu.sync_copy(seeds_hbm.at[pl.ds(c * SEEDS_PAD + off, SEED_CHUNK)], idx_v)
            pltpu.async_copy(h_hbm.at[c].at[idx_v], rows_v, sem).wait()
            pltpu.sync_copy(rows_v, out_hbm.at[c, pl.ds(off, SEED_CHUNK)])

    return k(h, seeds)


def _dinv_block(cnt_blk, blk_idx):
    dinv = lax.rsqrt(cnt_blk + 1.0)
    rows = blk_idx * BR + lax.broadcasted_iota(jnp.int32, (BR, 1), 0)
    return jnp.where(rows < N, dinv, 0.0)


def _tc_hn0(x, cnt):
    def body(x_ref, cnt_ref, hn_ref):
        dinv = _dinv_block(cnt_ref[0], pl.program_id(1))
        hn_ref[0] = x_ref[0] * dinv

    return pl.pallas_call(
        body,
        grid=(NC, NBLK),
        in_specs=[
            pl.BlockSpec((1, BR, D), lambda g, i: (g, i, 0)),
            pl.BlockSpec((1, BR, D), lambda g, i: (g, i, 0)),
        ],
        out_specs=pl.BlockSpec((1, BR, D), lambda g, i: (g, i, 0)),
        out_shape=jax.ShapeDtypeStruct((NC, NROWS, D), jnp.float32),
    )(x, cnt)


def _tc_layer(s_arr, hn, hp, cnt, w):
    def body(s_ref, hn_ref, hp_ref, cnt_ref, w_ref, ho_ref, hno_ref):
        dinv = _dinv_block(cnt_ref[0], pl.program_id(1))
        agg = (s_ref[0] + hn_ref[0]) * dinv
        h = jnp.dot(agg, w_ref[...], preferred_element_type=jnp.float32) + hp_ref[0]
        h = jnp.maximum(h, 0.0)
        ho_ref[0] = h
        hno_ref[0] = h * dinv

    return pl.pallas_call(
        body,
        grid=(NC, NBLK),
        in_specs=[
            pl.BlockSpec((1, BR, D), lambda g, i: (g, i, 0)),
            pl.BlockSpec((1, BR, D), lambda g, i: (g, i, 0)),
            pl.BlockSpec((1, BR, D), lambda g, i: (g, i, 0)),
            pl.BlockSpec((1, BR, D), lambda g, i: (g, i, 0)),
            pl.BlockSpec((D, D), lambda g, i: (0, 0)),
        ],
        out_specs=[
            pl.BlockSpec((1, BR, D), lambda g, i: (g, i, 0)),
            pl.BlockSpec((1, BR, D), lambda g, i: (g, i, 0)),
        ],
        out_shape=[
            jax.ShapeDtypeStruct((NC, NROWS, D), jnp.float32),
            jax.ShapeDtypeStruct((NC, NROWS, D), jnp.float32),
        ],
    )(s_arr, hn, hp, cnt, w)


def _pad_idx(a, n_pad, fill):
    return jnp.concatenate([a, jnp.full((n_pad - a.shape[0],), fill, jnp.int32)])


def kernel(sr_ent_seeds, tg_ent_seeds, edges_sr, edges_tg,
           entity_vec_sr, entity_vec_tg, W0, W1):
    src = jnp.concatenate([_pad_idx(edges_sr[0], E_PAD, N),
                           _pad_idx(edges_tg[0], E_PAD, N)])
    dst = jnp.concatenate([_pad_idx(edges_sr[1], E_PAD, N),
                           _pad_idx(edges_tg[1], E_PAD, N)])
    seeds = jnp.concatenate([_pad_idx(sr_ent_seeds, SEEDS_PAD, 0),
                             _pad_idx(tg_ent_seeds, SEEDS_PAD, 0)])
    x0 = jnp.stack([
        jnp.pad(entity_vec_sr, ((0, NROWS - N), (0, 0))),
        jnp.pad(entity_vec_tg, ((0, NROWS - N), (0, 0))),
    ])

    cnt = _sc_count(dst)
    hn0 = _tc_hn0(x0, cnt)
    s0 = _sc_scatter(hn0, src, dst)
    h1, hn1 = _tc_layer(s0, hn0, x0, cnt, W0)
    s1 = _sc_scatter(hn1, src, dst)
    h2, _ = _tc_layer(s1, hn1, h1, cnt, W1)
    sg = _sc_seed_gather(h2, seeds)

    return (sg[0, :NUM_SEEDS], sg[1, :NUM_SEEDS], h2[0, :N], h2[1, :N])
